# baseline (speedup 1.0000x reference)
import functools

import jax
import jax.numpy as jnp
from jax import lax
from jax.experimental import pallas as pl
from jax.experimental.pallas import tpu as pltpu
from jax.experimental.pallas import tpu_sc as plsc

VOCAB_SIZE = 1000000
N_EMBD = 64
SEQ_LEN = 2048
BATCH = 64

_NC = 2
_NS = 16
_NW = _NC * _NS

_B_FLAT = BATCH * SEQ_LEN
_B_PER_W = _B_FLAT // _NW
_CHUNK = 512
_N_CHUNKS = _B_PER_W // _CHUNK
_GATHERS = 4
_G = _CHUNK // _GATHERS


def _embed_kernel(x_ref, table_ref, pos_ref, out_ref,
                  idx_v, emb_v, pos_v, sem):
    wid = lax.axis_index("s") * _NC + lax.axis_index("c")
    base_w = wid * _B_PER_W

    for c in range(_N_CHUNKS):
        base = base_w + c * _CHUNK
        for j in range(_GATHERS):
            pltpu.sync_copy(x_ref.at[pl.ds(base + j * _G, _G)], idx_v.at[j])
        cps = [
            pltpu.async_copy(table_ref.at[idx_v.at[j]],
                             emb_v.at[pl.ds(j * _G, _G)], sem)
            for j in range(_GATHERS)
        ]
        pos_off = (c * _CHUNK) % SEQ_LEN
        pltpu.sync_copy(pos_ref.at[pl.ds(pos_off, _CHUNK)], pos_v)
        for cp in cps:
            cp.wait()

        @plsc.parallel_loop(0, _CHUNK, unroll=4)
        def _add_row(r):
            for j in range(N_EMBD // 16):
                sl = pl.ds(j * 16, 16)
                emb_v[r, sl] = emb_v[r, sl] + pos_v[r, sl]

        pltpu.sync_copy(emb_v, out_ref.at[pl.ds(base, _CHUNK)])


@jax.jit
def _embed(x_flat, token_embedding, pos_encoding):
    mesh = plsc.VectorSubcoreMesh(core_axis_name="c", subcore_axis_name="s")
    return pl.kernel(
        _embed_kernel,
        out_type=jax.ShapeDtypeStruct((_B_FLAT, N_EMBD), jnp.float32),
        mesh=mesh,
        scratch_types=[
            pltpu.VMEM((_GATHERS, _G), jnp.int32),
            pltpu.VMEM((_CHUNK, N_EMBD), jnp.float32),
            pltpu.VMEM((_CHUNK, N_EMBD), jnp.float32),
            pltpu.SemaphoreType.DMA,
        ],
        compiler_params=pltpu.CompilerParams(use_tc_tiling_on_sc=False),
    )(x_flat, token_embedding, pos_encoding)


def kernel(x, token_embedding, pos_encoding):
    x_flat = x.reshape(-1).astype(jnp.int32)
    out = _embed(x_flat, token_embedding, pos_encoding)
    return out.reshape(BATCH, SEQ_LEN, N_EMBD)


# speedup vs baseline: 1.0137x; 1.0137x over previous
import functools

import jax
import jax.numpy as jnp
from jax import lax
from jax.experimental import pallas as pl
from jax.experimental.pallas import tpu as pltpu
from jax.experimental.pallas import tpu_sc as plsc

VOCAB_SIZE = 1000000
N_EMBD = 64
SEQ_LEN = 2048
BATCH = 64

_NC = 2
_NS = 16
_NW = _NC * _NS

_B_FLAT = BATCH * SEQ_LEN
_B_PER_W = _B_FLAT // _NW
_CHUNK = 512
_N_CHUNKS = _B_PER_W // _CHUNK
_GATHERS = 4
_G = _CHUNK // _GATHERS


def _embed_kernel(x_ref, table_ref, pos_ref, out_ref,
                  idx_v, emb_v, sem):
    wid = lax.axis_index("s") * _NC + lax.axis_index("c")
    base_w = wid * _B_PER_W

    for c in range(_N_CHUNKS):
        base = base_w + c * _CHUNK
        for j in range(_GATHERS):
            pltpu.sync_copy(x_ref.at[pl.ds(base + j * _G, _G)], idx_v.at[j])
        pos_off = (c * _CHUNK) % SEQ_LEN
        pltpu.sync_copy(pos_ref.at[pl.ds(pos_off, _CHUNK)], emb_v)
        cps = [
            pltpu.async_copy(table_ref.at[idx_v.at[j]],
                             emb_v.at[pl.ds(j * _G, _G)], sem, add=True)
            for j in range(_GATHERS)
        ]
        for cp in cps:
            cp.wait()
        pltpu.sync_copy(emb_v, out_ref.at[pl.ds(base, _CHUNK)])


@jax.jit
def _embed(x_flat, token_embedding, pos_encoding):
    mesh = plsc.VectorSubcoreMesh(core_axis_name="c", subcore_axis_name="s")
    return pl.kernel(
        _embed_kernel,
        out_type=jax.ShapeDtypeStruct((_B_FLAT, N_EMBD), jnp.float32),
        mesh=mesh,
        scratch_types=[
            pltpu.VMEM((_GATHERS, _G), jnp.int32),
            pltpu.VMEM((_CHUNK, N_EMBD), jnp.float32),
            pltpu.SemaphoreType.DMA,
        ],
        compiler_params=pltpu.CompilerParams(use_tc_tiling_on_sc=False),
    )(x_flat, token_embedding, pos_encoding)


def kernel(x, token_embedding, pos_encoding):
    x_flat = x.reshape(-1).astype(jnp.int32)
    out = _embed(x_flat, token_embedding, pos_encoding)
    return out.reshape(BATCH, SEQ_LEN, N_EMBD)
